# Initial kernel scaffold; baseline (speedup 1.0000x reference)
import jax
import jax.numpy as jnp
from jax.experimental import pallas as pl


def kernel(x, W_gate):
    raise NotImplementedError("write your pallas kernel here")


# speedup vs baseline: 1.4004x; 1.4004x over previous
import functools

import jax
import jax.numpy as jnp
from jax.experimental import pallas as pl

_D_MODEL = 2048
_N_EXPERTS = 64
_BLK = 512


def _router_body(x_ref, w_ref, probs_ref, idx_ref, scores_ref):
    x = x_ref[...]
    w = w_ref[...]
    logits = jax.lax.dot_general(
        x, w, (((1,), (1,)), ((), ())), preferred_element_type=jnp.float32
    )

    m1 = jnp.max(logits, axis=-1, keepdims=True)
    p = jnp.exp(logits - m1)
    probs_ref[...] = p / jnp.sum(p, axis=-1, keepdims=True)

    iota = jax.lax.broadcasted_iota(jnp.int32, logits.shape, 1)
    i1 = jnp.min(jnp.where(logits == m1, iota, _N_EXPERTS), axis=-1)
    masked = jnp.where(iota == i1[:, None], -jnp.inf, logits)
    m2 = jnp.max(masked, axis=-1, keepdims=True)
    i2 = jnp.min(jnp.where(masked == m2, iota, _N_EXPERTS), axis=-1)
    idx_ref[...] = jnp.concatenate([i1[:, None], i2[:, None]], axis=-1)

    e2 = jnp.exp(m2 - m1)
    den = 1.0 + e2
    scores_ref[...] = jnp.concatenate([1.0 / den, e2 / den], axis=-1)


@functools.partial(jax.jit, static_argnames=())
def kernel(x, W_gate):
    b, s, d = x.shape
    tokens = b * s
    x2 = x.reshape(tokens, d)
    grid = (tokens // _BLK,)
    probs, idx, scores = pl.pallas_call(
        _router_body,
        grid=grid,
        in_specs=[
            pl.BlockSpec((_BLK, d), lambda i: (i, 0)),
            pl.BlockSpec((_N_EXPERTS, d), lambda i: (0, 0)),
        ],
        out_specs=[
            pl.BlockSpec((_BLK, _N_EXPERTS), lambda i: (i, 0)),
            pl.BlockSpec((_BLK, 2), lambda i: (i, 0)),
            pl.BlockSpec((_BLK, 2), lambda i: (i, 0)),
        ],
        out_shape=[
            jax.ShapeDtypeStruct((tokens, _N_EXPERTS), jnp.float32),
            jax.ShapeDtypeStruct((tokens, 2), jnp.int32),
            jax.ShapeDtypeStruct((tokens, 2), jnp.float32),
        ],
    )(x2, W_gate)
    return (
        idx.reshape(b, s, 2),
        scores.reshape(b, s, 2),
        probs.reshape(b, s, _N_EXPERTS),
    )


# speedup vs baseline: 1.5942x; 1.1384x over previous
import functools

import jax
import jax.numpy as jnp
from jax.experimental import pallas as pl

_D_MODEL = 2048
_N_EXPERTS = 64
_BLK = 1024


def _router_body(x_ref, w_ref, probs_ref, idx_ref, scores_ref):
    x = x_ref[...]
    w = w_ref[...]
    logits = jax.lax.dot_general(
        x, w, (((1,), (1,)), ((), ())), preferred_element_type=jnp.float32
    )

    m1 = jnp.max(logits, axis=-1, keepdims=True)
    p = jnp.exp(logits - m1)
    probs_ref[...] = p / jnp.sum(p, axis=-1, keepdims=True)

    iota = jax.lax.broadcasted_iota(jnp.int32, logits.shape, 1)
    i1 = jnp.min(jnp.where(logits == m1, iota, _N_EXPERTS), axis=-1)
    masked = jnp.where(iota == i1[:, None], -jnp.inf, logits)
    m2 = jnp.max(masked, axis=-1, keepdims=True)
    i2 = jnp.min(jnp.where(masked == m2, iota, _N_EXPERTS), axis=-1)
    idx_ref[...] = jnp.concatenate([i1[:, None], i2[:, None]], axis=-1)

    e2 = jnp.exp(m2 - m1)
    den = 1.0 + e2
    scores_ref[...] = jnp.concatenate([1.0 / den, e2 / den], axis=-1)


@functools.partial(jax.jit, static_argnames=())
def kernel(x, W_gate):
    b, s, d = x.shape
    tokens = b * s
    x2 = x.reshape(tokens, d)
    grid = (tokens // _BLK,)
    probs, idx, scores = pl.pallas_call(
        _router_body,
        grid=grid,
        in_specs=[
            pl.BlockSpec((_BLK, d), lambda i: (i, 0)),
            pl.BlockSpec((_N_EXPERTS, d), lambda i: (0, 0)),
        ],
        out_specs=[
            pl.BlockSpec((_BLK, _N_EXPERTS), lambda i: (i, 0)),
            pl.BlockSpec((_BLK, 2), lambda i: (i, 0)),
            pl.BlockSpec((_BLK, 2), lambda i: (i, 0)),
        ],
        out_shape=[
            jax.ShapeDtypeStruct((tokens, _N_EXPERTS), jnp.float32),
            jax.ShapeDtypeStruct((tokens, 2), jnp.int32),
            jax.ShapeDtypeStruct((tokens, 2), jnp.float32),
        ],
    )(x2, W_gate)
    return (
        idx.reshape(b, s, 2),
        scores.reshape(b, s, 2),
        probs.reshape(b, s, _N_EXPERTS),
    )


# speedup vs baseline: 1.6509x; 1.0356x over previous
import functools

import jax
import jax.numpy as jnp
from jax.experimental import pallas as pl

_D_MODEL = 2048
_N_EXPERTS = 64
_BLK = 2048


def _router_body(x_ref, w_ref, probs_ref, idx_ref, scores_ref):
    x = x_ref[...]
    w = w_ref[...]
    logits = jax.lax.dot_general(
        x, w, (((1,), (1,)), ((), ())), preferred_element_type=jnp.float32
    )

    m1 = jnp.max(logits, axis=-1, keepdims=True)
    p = jnp.exp(logits - m1)
    probs_ref[...] = p / jnp.sum(p, axis=-1, keepdims=True)

    iota = jax.lax.broadcasted_iota(jnp.int32, logits.shape, 1)
    i1 = jnp.min(jnp.where(logits == m1, iota, _N_EXPERTS), axis=-1)
    masked = jnp.where(iota == i1[:, None], -jnp.inf, logits)
    m2 = jnp.max(masked, axis=-1, keepdims=True)
    i2 = jnp.min(jnp.where(masked == m2, iota, _N_EXPERTS), axis=-1)
    idx_ref[...] = jnp.concatenate([i1[:, None], i2[:, None]], axis=-1)

    e2 = jnp.exp(m2 - m1)
    den = 1.0 + e2
    scores_ref[...] = jnp.concatenate([1.0 / den, e2 / den], axis=-1)


@functools.partial(jax.jit, static_argnames=())
def kernel(x, W_gate):
    b, s, d = x.shape
    tokens = b * s
    x2 = x.reshape(tokens, d)
    grid = (tokens // _BLK,)
    probs, idx, scores = pl.pallas_call(
        _router_body,
        grid=grid,
        in_specs=[
            pl.BlockSpec((_BLK, d), lambda i: (i, 0)),
            pl.BlockSpec((_N_EXPERTS, d), lambda i: (0, 0)),
        ],
        out_specs=[
            pl.BlockSpec((_BLK, _N_EXPERTS), lambda i: (i, 0)),
            pl.BlockSpec((_BLK, 2), lambda i: (i, 0)),
            pl.BlockSpec((_BLK, 2), lambda i: (i, 0)),
        ],
        out_shape=[
            jax.ShapeDtypeStruct((tokens, _N_EXPERTS), jnp.float32),
            jax.ShapeDtypeStruct((tokens, 2), jnp.int32),
            jax.ShapeDtypeStruct((tokens, 2), jnp.float32),
        ],
    )(x2, W_gate)
    return (
        idx.reshape(b, s, 2),
        scores.reshape(b, s, 2),
        probs.reshape(b, s, _N_EXPERTS),
    )
